# baseline (speedup 1.0000x reference)
import jax
import jax.numpy as jnp
from jax.experimental import pallas as pl
from jax.experimental.pallas import tpu as pltpu

SEQ = 2048
D_MODEL = 768
N_CA3 = 4096
K_WTA = 64
TN = 256
GRID = SEQ // TN


def _kwta_normalize(hb):
    n, ncols = hb.shape
    thr = jnp.zeros((1, ncols), dtype=jnp.int32)
    for b in range(30, 15, -1):
        cand = thr | (1 << b)
        cand_bf = jax.lax.bitcast_convert_type(cand, jnp.float32).astype(
            jnp.bfloat16)
        m = (hb >= cand_bf).astype(jnp.bfloat16)
        part = jnp.sum(m.reshape(16, n // 16, ncols), axis=1)
        cnt = jnp.sum(part.astype(jnp.float32), axis=0, keepdims=True)
        thr = jnp.where(cnt >= K_WTA, cand, thr)
    thr_bf = jax.lax.bitcast_convert_type(thr, jnp.float32).astype(
        jnp.bfloat16)
    sb = jnp.where(hb >= thr_bf, hb, jnp.bfloat16(0))
    sq = sb * sb
    p2 = jnp.sum(sq.reshape(16, n // 16, ncols), axis=1)
    ns = jnp.sum(p2.astype(jnp.float32), axis=0, keepdims=True)
    inv = (1.0 / jnp.maximum(jnp.sqrt(ns), 1e-10)).astype(jnp.bfloat16)
    return sb * inv


def _sparsify_body(wd_ref, xT_ref, sT_ref):
    h = jnp.dot(wd_ref[...], xT_ref[...].astype(jnp.bfloat16),
                preferred_element_type=jnp.float32)
    hb = jnp.maximum(h, 0.0).astype(jnp.bfloat16)
    sT_ref[...] = _kwta_normalize(hb)


def _retrieve_body(wc_ref, sT_ref, h2T_ref, tot_ref, acc_ref):
    i = pl.program_id(0)
    k = pl.program_id(1)
    nk = pl.num_programs(1)

    part = jnp.dot(wc_ref[...], sT_ref[...],
                   preferred_element_type=jnp.float32)

    @pl.when(k == 0)
    def _init_acc():
        acc_ref[...] = part

    @pl.when(k != 0)
    def _accum():
        acc_ref[...] += part

    @pl.when((i == 0) & (k == 0))
    def _init_tot():
        tot_ref[...] = jnp.zeros((1, 1), jnp.float32)

    @pl.when(k == nk - 1)
    def _finish():
        h2 = jnp.maximum(acc_ref[...], 0.0)
        tot_ref[...] += jnp.sum(h2).reshape(1, 1)
        h2T_ref[...] = h2.astype(jnp.bfloat16)


def _combine_body(xT_ref, sT_ref, h2T_ref, tot_ref, wu_ref, wg_ref, bg_ref,
                  outT_ref):
    xn = _kwta_normalize(h2T_ref[...])
    cond = tot_ref[...] < 1e-10
    succ = jnp.where(cond, sT_ref[...], xn)
    predT = jnp.dot(wu_ref[...], succ,
                    preferred_element_type=jnp.float32)
    xT = xT_ref[...]
    zT = jnp.dot(wg_ref[...], xT.astype(jnp.bfloat16),
                 preferred_element_type=jnp.float32) + bg_ref[...]
    gT = jax.nn.sigmoid(zT)
    outT_ref[...] = xT + gT * predT


def kernel(x, W_down, W_up, W_gate, b_gate, W_ca3):
    xT = x.reshape(SEQ, D_MODEL).T
    wd = W_down.astype(jnp.bfloat16)
    wc = W_ca3.astype(jnp.bfloat16)
    wu = W_up.astype(jnp.bfloat16)
    wg = W_gate.astype(jnp.bfloat16)
    bg = b_gate.reshape(D_MODEL, 1)

    cp = pltpu.CompilerParams(vmem_limit_bytes=63 * 1024 * 1024)

    sT = pl.pallas_call(
        _sparsify_body,
        grid=(GRID,),
        in_specs=[
            pl.BlockSpec((N_CA3, D_MODEL), lambda i: (0, 0)),
            pl.BlockSpec((D_MODEL, TN), lambda i: (0, i)),
        ],
        out_specs=pl.BlockSpec((N_CA3, TN), lambda i: (0, i)),
        out_shape=jax.ShapeDtypeStruct((N_CA3, SEQ), jnp.bfloat16),
        compiler_params=cp,
    )(wd, xT)

    KP = 1024
    h2T, tot = pl.pallas_call(
        _retrieve_body,
        grid=(GRID, N_CA3 // KP),
        in_specs=[
            pl.BlockSpec((N_CA3, KP), lambda i, k: (0, k)),
            pl.BlockSpec((KP, TN), lambda i, k: (k, i)),
        ],
        out_specs=[
            pl.BlockSpec((N_CA3, TN), lambda i, k: (0, i)),
            pl.BlockSpec((1, 1), lambda i, k: (0, 0)),
        ],
        out_shape=[
            jax.ShapeDtypeStruct((N_CA3, SEQ), jnp.bfloat16),
            jax.ShapeDtypeStruct((1, 1), jnp.float32),
        ],
        scratch_shapes=[
            pltpu.VMEM((N_CA3, TN), jnp.float32),
        ],
        compiler_params=cp,
    )(wc, sT)

    outT = pl.pallas_call(
        _combine_body,
        grid=(GRID,),
        in_specs=[
            pl.BlockSpec((D_MODEL, TN), lambda i: (0, i)),
            pl.BlockSpec((N_CA3, TN), lambda i: (0, i)),
            pl.BlockSpec((N_CA3, TN), lambda i: (0, i)),
            pl.BlockSpec((1, 1), lambda i: (0, 0)),
            pl.BlockSpec((D_MODEL, N_CA3), lambda i: (0, 0)),
            pl.BlockSpec((D_MODEL, D_MODEL), lambda i: (0, 0)),
            pl.BlockSpec((D_MODEL, 1), lambda i: (0, 0)),
        ],
        out_specs=pl.BlockSpec((D_MODEL, TN), lambda i: (0, i)),
        out_shape=jax.ShapeDtypeStruct((D_MODEL, SEQ), jnp.float32),
        compiler_params=cp,
    )(xT, sT, h2T, tot, wu, wg, bg)

    return outT.T.reshape(1, SEQ, D_MODEL)


# speedup vs baseline: 1.5620x; 1.5620x over previous
import jax
import jax.numpy as jnp
from jax.experimental import pallas as pl
from jax.experimental.pallas import tpu as pltpu

SEQ = 2048
D_MODEL = 768
N_CA3 = 4096
K_WTA = 64
TN = 256
GRID = SEQ // TN


def _kwta_normalize(h):
    ncols = h.shape[1]
    thr = jnp.zeros((1, ncols), dtype=jnp.int32)
    for b in range(30, 17, -1):
        cand = thr | (1 << b)
        cand_f = jax.lax.bitcast_convert_type(cand, jnp.float32)
        cnt = jnp.sum((h >= cand_f).astype(jnp.int32), axis=0, keepdims=True)
        thr = jnp.where(cnt >= K_WTA, cand, thr)
    thr_f = jax.lax.bitcast_convert_type(thr, jnp.float32)
    s = jnp.where(h >= thr_f, h, 0.0)
    norm = jnp.sqrt(jnp.sum(s * s, axis=0, keepdims=True))
    return s * (1.0 / jnp.maximum(norm, 1e-10))


def _sparsify_body(wd_ref, xT_ref, sT_ref):
    h = jnp.dot(wd_ref[...], xT_ref[...].astype(jnp.bfloat16),
                preferred_element_type=jnp.float32)
    h = jnp.maximum(h, 0.0)
    sT_ref[...] = _kwta_normalize(h).astype(jnp.bfloat16)


def _retrieve_body(wc_ref, sT_ref, h2T_ref, tot_ref, acc_ref):
    i = pl.program_id(0)
    k = pl.program_id(1)
    nk = pl.num_programs(1)

    part = jnp.dot(wc_ref[...], sT_ref[...],
                   preferred_element_type=jnp.float32)

    @pl.when(k == 0)
    def _init_acc():
        acc_ref[...] = part

    @pl.when(k != 0)
    def _accum():
        acc_ref[...] += part

    @pl.when((i == 0) & (k == 0))
    def _init_tot():
        tot_ref[...] = jnp.zeros((1, 1), jnp.float32)

    @pl.when(k == nk - 1)
    def _finish():
        h2 = jnp.maximum(acc_ref[...] * (1.0 / 1024.0), 0.0)
        tot_ref[...] += jnp.sum(h2).reshape(1, 1)
        h2T_ref[...] = h2.astype(jnp.bfloat16)


def _combine_body(xT_ref, sT_ref, h2T_ref, tot_ref, wu_ref, wg_ref, bg_ref,
                  outT_ref):
    xn = _kwta_normalize(h2T_ref[...].astype(jnp.float32))
    cond = tot_ref[...] < 1e-10
    succ = jnp.where(cond, sT_ref[...].astype(jnp.float32), xn)
    predT = jnp.dot(wu_ref[...], succ.astype(jnp.bfloat16),
                    preferred_element_type=jnp.float32)
    xT = xT_ref[...]
    zT = jnp.dot(wg_ref[...], xT.astype(jnp.bfloat16),
                 preferred_element_type=jnp.float32) + bg_ref[...]
    gT = jax.nn.sigmoid(zT)
    outT_ref[...] = xT + gT * predT


def kernel(x, W_down, W_up, W_gate, b_gate, W_ca3):
    xT = x.reshape(SEQ, D_MODEL).T
    wd = W_down.astype(jnp.bfloat16)
    wc = (W_ca3 * 64.0).astype(jnp.float8_e4m3fn)
    wu = W_up.astype(jnp.bfloat16)
    wg = W_gate.astype(jnp.bfloat16)
    bg = b_gate.reshape(D_MODEL, 1)

    cp = pltpu.CompilerParams(vmem_limit_bytes=63 * 1024 * 1024)

    sT = pl.pallas_call(
        _sparsify_body,
        grid=(GRID,),
        in_specs=[
            pl.BlockSpec((N_CA3, D_MODEL), lambda i: (0, 0)),
            pl.BlockSpec((D_MODEL, TN), lambda i: (0, i)),
        ],
        out_specs=pl.BlockSpec((N_CA3, TN), lambda i: (0, i)),
        out_shape=jax.ShapeDtypeStruct((N_CA3, SEQ), jnp.bfloat16),
        compiler_params=cp,
    )(wd, xT)

    s8 = (sT.astype(jnp.float32) * 16.0).astype(jnp.float8_e4m3fn)

    KP = 2048
    h2T, tot = pl.pallas_call(
        _retrieve_body,
        grid=(GRID, N_CA3 // KP),
        in_specs=[
            pl.BlockSpec((N_CA3, KP), lambda i, k: (0, k)),
            pl.BlockSpec((KP, TN), lambda i, k: (k, i)),
        ],
        out_specs=[
            pl.BlockSpec((N_CA3, TN), lambda i, k: (0, i)),
            pl.BlockSpec((1, 1), lambda i, k: (0, 0)),
        ],
        out_shape=[
            jax.ShapeDtypeStruct((N_CA3, SEQ), jnp.bfloat16),
            jax.ShapeDtypeStruct((1, 1), jnp.float32),
        ],
        scratch_shapes=[
            pltpu.VMEM((N_CA3, TN), jnp.float32),
        ],
        compiler_params=cp,
    )(wc, s8)

    outT = pl.pallas_call(
        _combine_body,
        grid=(GRID,),
        in_specs=[
            pl.BlockSpec((D_MODEL, TN), lambda i: (0, i)),
            pl.BlockSpec((N_CA3, TN), lambda i: (0, i)),
            pl.BlockSpec((N_CA3, TN), lambda i: (0, i)),
            pl.BlockSpec((1, 1), lambda i: (0, 0)),
            pl.BlockSpec((D_MODEL, N_CA3), lambda i: (0, 0)),
            pl.BlockSpec((D_MODEL, D_MODEL), lambda i: (0, 0)),
            pl.BlockSpec((D_MODEL, 1), lambda i: (0, 0)),
        ],
        out_specs=pl.BlockSpec((D_MODEL, TN), lambda i: (0, i)),
        out_shape=jax.ShapeDtypeStruct((D_MODEL, SEQ), jnp.float32),
        compiler_params=cp,
    )(xT, sT, h2T, tot, wu, wg, bg)

    return outT.T.reshape(1, SEQ, D_MODEL)


# speedup vs baseline: 1.5750x; 1.0083x over previous
import jax
import jax.numpy as jnp
from jax.experimental import pallas as pl
from jax.experimental.pallas import tpu as pltpu

SEQ = 2048
D_MODEL = 768
N_CA3 = 4096
K_WTA = 64
TN = 256
GRID = SEQ // TN


def _kwta_normalize(h):
    ncols = h.shape[1]
    thr = jnp.zeros((1, ncols), dtype=jnp.int32)
    for b in range(30, 17, -1):
        cand = thr | (1 << b)
        cand_f = jax.lax.bitcast_convert_type(cand, jnp.float32)
        cnt = jnp.sum((h >= cand_f).astype(jnp.int32), axis=0, keepdims=True)
        thr = jnp.where(cnt >= K_WTA, cand, thr)
    thr_f = jax.lax.bitcast_convert_type(thr, jnp.float32)
    s = jnp.where(h >= thr_f, h, 0.0)
    norm = jnp.sqrt(jnp.sum(s * s, axis=0, keepdims=True))
    return s * (1.0 / jnp.maximum(norm, 1e-10))


def _sparsify_body(wd_ref, xT_ref, sT_ref):
    h = jnp.dot(wd_ref[...], xT_ref[...].astype(jnp.bfloat16),
                preferred_element_type=jnp.float32)
    h = jnp.maximum(h, 0.0)
    sT_ref[...] = (_kwta_normalize(h) * 16.0).astype(jnp.float8_e4m3fn)


def _retrieve_body(wc_ref, sT_ref, h2T_ref, tot_ref, acc_ref):
    i = pl.program_id(0)
    k = pl.program_id(1)
    nk = pl.num_programs(1)

    part = jnp.dot(wc_ref[...], sT_ref[...],
                   preferred_element_type=jnp.float32)

    @pl.when(k == 0)
    def _init_acc():
        acc_ref[...] = part

    @pl.when(k != 0)
    def _accum():
        acc_ref[...] += part

    @pl.when((i == 0) & (k == 0))
    def _init_tot():
        tot_ref[...] = jnp.zeros((1, 1), jnp.float32)

    @pl.when(k == nk - 1)
    def _finish():
        h2 = jnp.maximum(acc_ref[...] * (1.0 / 1024.0), 0.0)
        tot_ref[...] += jnp.sum(h2).reshape(1, 1)
        h2T_ref[...] = h2.astype(jnp.bfloat16)


def _combine_body(xT_ref, sT_ref, h2T_ref, tot_ref, wu_ref, wg_ref, bg_ref,
                  outT_ref):
    xn = _kwta_normalize(h2T_ref[...].astype(jnp.float32))
    cond = tot_ref[...] < 1e-10
    succ = jnp.where(cond, sT_ref[...].astype(jnp.float32) * (1.0 / 16.0), xn)
    predT = jnp.dot(wu_ref[...], succ.astype(jnp.bfloat16),
                    preferred_element_type=jnp.float32)
    xT = xT_ref[...]
    zT = jnp.dot(wg_ref[...], xT.astype(jnp.bfloat16),
                 preferred_element_type=jnp.float32) + bg_ref[...]
    gT = jax.nn.sigmoid(zT)
    outT_ref[...] = xT + gT * predT


def kernel(x, W_down, W_up, W_gate, b_gate, W_ca3):
    xT = x.reshape(SEQ, D_MODEL).T
    wd = W_down.astype(jnp.bfloat16)
    wc = (W_ca3 * 64.0).astype(jnp.float8_e4m3fn)
    wu = W_up.astype(jnp.bfloat16)
    wg = W_gate.astype(jnp.bfloat16)
    bg = b_gate.reshape(D_MODEL, 1)

    cp = pltpu.CompilerParams(vmem_limit_bytes=63 * 1024 * 1024)

    sT = pl.pallas_call(
        _sparsify_body,
        grid=(GRID,),
        in_specs=[
            pl.BlockSpec((N_CA3, D_MODEL), lambda i: (0, 0)),
            pl.BlockSpec((D_MODEL, TN), lambda i: (0, i)),
        ],
        out_specs=pl.BlockSpec((N_CA3, TN), lambda i: (0, i)),
        out_shape=jax.ShapeDtypeStruct((N_CA3, SEQ), jnp.float8_e4m3fn),
        compiler_params=cp,
    )(wd, xT)

    KP = 2048
    h2T, tot = pl.pallas_call(
        _retrieve_body,
        grid=(GRID, N_CA3 // KP),
        in_specs=[
            pl.BlockSpec((N_CA3, KP), lambda i, k: (0, k)),
            pl.BlockSpec((KP, TN), lambda i, k: (k, i)),
        ],
        out_specs=[
            pl.BlockSpec((N_CA3, TN), lambda i, k: (0, i)),
            pl.BlockSpec((1, 1), lambda i, k: (0, 0)),
        ],
        out_shape=[
            jax.ShapeDtypeStruct((N_CA3, SEQ), jnp.bfloat16),
            jax.ShapeDtypeStruct((1, 1), jnp.float32),
        ],
        scratch_shapes=[
            pltpu.VMEM((N_CA3, TN), jnp.float32),
        ],
        compiler_params=cp,
    )(wc, sT)

    outT = pl.pallas_call(
        _combine_body,
        grid=(GRID,),
        in_specs=[
            pl.BlockSpec((D_MODEL, TN), lambda i: (0, i)),
            pl.BlockSpec((N_CA3, TN), lambda i: (0, i)),
            pl.BlockSpec((N_CA3, TN), lambda i: (0, i)),
            pl.BlockSpec((1, 1), lambda i: (0, 0)),
            pl.BlockSpec((D_MODEL, N_CA3), lambda i: (0, 0)),
            pl.BlockSpec((D_MODEL, D_MODEL), lambda i: (0, 0)),
            pl.BlockSpec((D_MODEL, 1), lambda i: (0, 0)),
        ],
        out_specs=pl.BlockSpec((D_MODEL, TN), lambda i: (0, i)),
        out_shape=jax.ShapeDtypeStruct((D_MODEL, SEQ), jnp.float32),
        compiler_params=cp,
    )(xT, sT, h2T, tot, wu, wg, bg)

    return outT.T.reshape(1, SEQ, D_MODEL)


# speedup vs baseline: 1.6128x; 1.0240x over previous
import jax
import jax.numpy as jnp
from jax.experimental import pallas as pl
from jax.experimental.pallas import tpu as pltpu

SEQ = 2048
D_MODEL = 768
N_CA3 = 4096
K_WTA = 64
TN = 256
GRID = SEQ // TN


def _kwta_normalize(h):
    ncols = h.shape[1]
    thr = jnp.zeros((1, ncols), dtype=jnp.int32)
    for b in range(30, 17, -1):
        cand = thr | (1 << b)
        cand_f = jax.lax.bitcast_convert_type(cand, jnp.float32)
        cnt = jnp.sum((h >= cand_f).astype(jnp.int32), axis=0, keepdims=True)
        thr = jnp.where(cnt >= K_WTA, cand, thr)
    thr_f = jax.lax.bitcast_convert_type(thr, jnp.float32)
    s = jnp.where(h >= thr_f, h, 0.0)
    norm = jnp.sqrt(jnp.sum(s * s, axis=0, keepdims=True))
    return s * (1.0 / jnp.maximum(norm, 1e-10))


def _sparsify_body(wd_ref, xT_ref, sT_ref, h_ref):
    i = pl.program_id(0)
    cur = jax.lax.rem(i, 2)

    @pl.when(i < GRID)
    def _proj():
        h = jnp.dot(wd_ref[...], xT_ref[...].astype(jnp.bfloat16),
                    preferred_element_type=jnp.float32)
        h_ref[cur] = jnp.maximum(h, 0.0)

    @pl.when(i > 0)
    def _kwta():
        sT_ref[...] = (_kwta_normalize(h_ref[1 - cur]) * 16.0).astype(
            jnp.float8_e4m3fn)


def _retrieve_body(wc_ref, sT_ref, h2T_ref, tot_ref, acc_ref):
    i = pl.program_id(0)
    k = pl.program_id(1)
    nk = pl.num_programs(1)

    part = jnp.dot(wc_ref[...], sT_ref[...],
                   preferred_element_type=jnp.float32)

    @pl.when(k == 0)
    def _init_acc():
        acc_ref[...] = part

    @pl.when(k != 0)
    def _accum():
        acc_ref[...] += part

    @pl.when((i == 0) & (k == 0))
    def _init_tot():
        tot_ref[...] = jnp.zeros((1, 1), jnp.float32)

    @pl.when(k == nk - 1)
    def _finish():
        h2 = jnp.maximum(acc_ref[...] * (1.0 / 1024.0), 0.0)
        tot_ref[...] += jnp.sum(h2).reshape(1, 1)
        h2T_ref[...] = h2.astype(jnp.bfloat16)


def _combine_body(xT_ref, sT_ref, h2T_ref, tot_ref, wu_ref, wg_ref, bg_ref,
                  outT_ref):
    xn = _kwta_normalize(h2T_ref[...].astype(jnp.float32))
    cond = tot_ref[...] < 1e-10
    succ = jnp.where(cond, sT_ref[...].astype(jnp.float32) * (1.0 / 16.0), xn)
    predT = jnp.dot(wu_ref[...], succ.astype(jnp.bfloat16),
                    preferred_element_type=jnp.float32)
    xT = xT_ref[...]
    zT = jnp.dot(wg_ref[...], xT.astype(jnp.bfloat16),
                 preferred_element_type=jnp.float32) + bg_ref[...]
    gT = jax.nn.sigmoid(zT)
    outT_ref[...] = xT + gT * predT


def kernel(x, W_down, W_up, W_gate, b_gate, W_ca3):
    xT = x.reshape(SEQ, D_MODEL).T
    wd = W_down.astype(jnp.bfloat16)
    wc = (W_ca3 * 64.0).astype(jnp.float8_e4m3fn)
    wu = W_up.astype(jnp.bfloat16)
    wg = W_gate.astype(jnp.bfloat16)
    bg = b_gate.reshape(D_MODEL, 1)

    cp = pltpu.CompilerParams(vmem_limit_bytes=63 * 1024 * 1024)

    sT = pl.pallas_call(
        _sparsify_body,
        grid=(GRID + 1,),
        in_specs=[
            pl.BlockSpec((N_CA3, D_MODEL), lambda i: (0, 0)),
            pl.BlockSpec((D_MODEL, TN),
                         lambda i: (0, jnp.minimum(i, GRID - 1))),
        ],
        out_specs=pl.BlockSpec((N_CA3, TN),
                               lambda i: (0, jnp.maximum(i - 1, 0))),
        out_shape=jax.ShapeDtypeStruct((N_CA3, SEQ), jnp.float8_e4m3fn),
        scratch_shapes=[
            pltpu.VMEM((2, N_CA3, TN), jnp.float32),
        ],
        compiler_params=cp,
    )(wd, xT)

    KP = 2048
    h2T, tot = pl.pallas_call(
        _retrieve_body,
        grid=(GRID, N_CA3 // KP),
        in_specs=[
            pl.BlockSpec((N_CA3, KP), lambda i, k: (0, k)),
            pl.BlockSpec((KP, TN), lambda i, k: (k, i)),
        ],
        out_specs=[
            pl.BlockSpec((N_CA3, TN), lambda i, k: (0, i)),
            pl.BlockSpec((1, 1), lambda i, k: (0, 0)),
        ],
        out_shape=[
            jax.ShapeDtypeStruct((N_CA3, SEQ), jnp.bfloat16),
            jax.ShapeDtypeStruct((1, 1), jnp.float32),
        ],
        scratch_shapes=[
            pltpu.VMEM((N_CA3, TN), jnp.float32),
        ],
        compiler_params=cp,
    )(wc, sT)

    outT = pl.pallas_call(
        _combine_body,
        grid=(GRID,),
        in_specs=[
            pl.BlockSpec((D_MODEL, TN), lambda i: (0, i)),
            pl.BlockSpec((N_CA3, TN), lambda i: (0, i)),
            pl.BlockSpec((N_CA3, TN), lambda i: (0, i)),
            pl.BlockSpec((1, 1), lambda i: (0, 0)),
            pl.BlockSpec((D_MODEL, N_CA3), lambda i: (0, 0)),
            pl.BlockSpec((D_MODEL, D_MODEL), lambda i: (0, 0)),
            pl.BlockSpec((D_MODEL, 1), lambda i: (0, 0)),
        ],
        out_specs=pl.BlockSpec((D_MODEL, TN), lambda i: (0, i)),
        out_shape=jax.ShapeDtypeStruct((D_MODEL, SEQ), jnp.float32),
        compiler_params=cp,
    )(xT, sT, h2T, tot, wu, wg, bg)

    return outT.T.reshape(1, SEQ, D_MODEL)


# speedup vs baseline: 1.6223x; 1.0059x over previous
import jax
import jax.numpy as jnp
from jax.experimental import pallas as pl
from jax.experimental.pallas import tpu as pltpu

SEQ = 2048
D_MODEL = 768
N_CA3 = 4096
K_WTA = 64
TN = 256
GRID = SEQ // TN


def _kwta_normalize(h):
    ncols = h.shape[1]
    thr = jnp.zeros((1, ncols), dtype=jnp.int32)
    for b in range(30, 17, -1):
        cand = thr | (1 << b)
        cand_f = jax.lax.bitcast_convert_type(cand, jnp.float32)
        cnt = jnp.sum((h >= cand_f).astype(jnp.int32), axis=0, keepdims=True)
        thr = jnp.where(cnt >= K_WTA, cand, thr)
    thr_f = jax.lax.bitcast_convert_type(thr, jnp.float32)
    s = jnp.where(h >= thr_f, h, 0.0)
    norm = jnp.sqrt(jnp.sum(s * s, axis=0, keepdims=True))
    return s * (1.0 / jnp.maximum(norm, 1e-10))


def _sparsify_body(wd_ref, xT_ref, sT_ref, hprev_ref, hcur_ref):
    sT_ref[...] = (_kwta_normalize(hprev_ref[...]) * 16.0).astype(
        jnp.float8_e4m3fn)
    h = jnp.dot(wd_ref[...], xT_ref[...].astype(jnp.bfloat16),
                preferred_element_type=jnp.float32)
    hcur_ref[...] = jnp.maximum(h, 0.0)
    hprev_ref[...] = hcur_ref[...]


def _retrieve_body(wc_ref, sT_ref, h2T_ref, tot_ref, acc_ref):
    i = pl.program_id(0)
    k = pl.program_id(1)
    nk = pl.num_programs(1)

    part = jnp.dot(wc_ref[...], sT_ref[...],
                   preferred_element_type=jnp.float32)

    @pl.when(k == 0)
    def _init_acc():
        acc_ref[...] = part

    @pl.when(k != 0)
    def _accum():
        acc_ref[...] += part

    @pl.when((i == 0) & (k == 0))
    def _init_tot():
        tot_ref[...] = jnp.zeros((1, 1), jnp.float32)

    @pl.when(k == nk - 1)
    def _finish():
        h2 = jnp.maximum(acc_ref[...] * (1.0 / 1024.0), 0.0)
        tot_ref[...] += jnp.sum(h2).reshape(1, 1)
        h2T_ref[...] = h2.astype(jnp.bfloat16)


def _combine_body(xT_ref, sT_ref, h2T_ref, tot_ref, wu_ref, wg_ref, bg_ref,
                  outT_ref):
    xn = _kwta_normalize(h2T_ref[...].astype(jnp.float32))
    cond = tot_ref[...] < 1e-10
    succ = jnp.where(cond, sT_ref[...].astype(jnp.float32) * (1.0 / 16.0), xn)
    predT = jnp.dot(wu_ref[...], succ.astype(jnp.bfloat16),
                    preferred_element_type=jnp.float32)
    xT = xT_ref[...]
    zT = jnp.dot(wg_ref[...], xT.astype(jnp.bfloat16),
                 preferred_element_type=jnp.float32) + bg_ref[...]
    gT = jax.nn.sigmoid(zT)
    outT_ref[...] = xT + gT * predT


def kernel(x, W_down, W_up, W_gate, b_gate, W_ca3):
    xT = x.reshape(SEQ, D_MODEL).T
    wd = W_down.astype(jnp.bfloat16)
    wc = (W_ca3 * 64.0).astype(jnp.float8_e4m3fn)
    wu = W_up.astype(jnp.bfloat16)
    wg = W_gate.astype(jnp.bfloat16)
    bg = b_gate.reshape(D_MODEL, 1)

    cp = pltpu.CompilerParams(vmem_limit_bytes=63 * 1024 * 1024)

    sT = pl.pallas_call(
        _sparsify_body,
        grid=(GRID + 1,),
        in_specs=[
            pl.BlockSpec((N_CA3, D_MODEL), lambda i: (0, 0)),
            pl.BlockSpec((D_MODEL, TN),
                         lambda i: (0, jnp.minimum(i, GRID - 1))),
        ],
        out_specs=pl.BlockSpec((N_CA3, TN),
                               lambda i: (0, jnp.maximum(i - 1, 0))),
        out_shape=jax.ShapeDtypeStruct((N_CA3, SEQ), jnp.float8_e4m3fn),
        scratch_shapes=[
            pltpu.VMEM((N_CA3, TN), jnp.float32),
            pltpu.VMEM((N_CA3, TN), jnp.float32),
        ],
        compiler_params=cp,
    )(wd, xT)

    KP = 2048
    h2T, tot = pl.pallas_call(
        _retrieve_body,
        grid=(GRID, N_CA3 // KP),
        in_specs=[
            pl.BlockSpec((N_CA3, KP), lambda i, k: (0, k)),
            pl.BlockSpec((KP, TN), lambda i, k: (k, i)),
        ],
        out_specs=[
            pl.BlockSpec((N_CA3, TN), lambda i, k: (0, i)),
            pl.BlockSpec((1, 1), lambda i, k: (0, 0)),
        ],
        out_shape=[
            jax.ShapeDtypeStruct((N_CA3, SEQ), jnp.bfloat16),
            jax.ShapeDtypeStruct((1, 1), jnp.float32),
        ],
        scratch_shapes=[
            pltpu.VMEM((N_CA3, TN), jnp.float32),
        ],
        compiler_params=cp,
    )(wc, sT)

    outT = pl.pallas_call(
        _combine_body,
        grid=(GRID,),
        in_specs=[
            pl.BlockSpec((D_MODEL, TN), lambda i: (0, i)),
            pl.BlockSpec((N_CA3, TN), lambda i: (0, i)),
            pl.BlockSpec((N_CA3, TN), lambda i: (0, i)),
            pl.BlockSpec((1, 1), lambda i: (0, 0)),
            pl.BlockSpec((D_MODEL, N_CA3), lambda i: (0, 0)),
            pl.BlockSpec((D_MODEL, D_MODEL), lambda i: (0, 0)),
            pl.BlockSpec((D_MODEL, 1), lambda i: (0, 0)),
        ],
        out_specs=pl.BlockSpec((D_MODEL, TN), lambda i: (0, i)),
        out_shape=jax.ShapeDtypeStruct((D_MODEL, SEQ), jnp.float32),
        compiler_params=cp,
    )(xT, sT, h2T, tot, wu, wg, bg)

    return outT.T.reshape(1, SEQ, D_MODEL)


# speedup vs baseline: 1.7330x; 1.0682x over previous
import jax
import jax.numpy as jnp
from jax.experimental import pallas as pl
from jax.experimental.pallas import tpu as pltpu

SEQ = 2048
D_MODEL = 768
N_CA3 = 4096
K_WTA = 64
TN = 256
GRID = SEQ // TN


def _kwta_normalize(h):
    ncols = h.shape[1]
    thr = jnp.zeros((1, ncols), dtype=jnp.int32)
    for b in range(30, 19, -1):
        cand = thr | (1 << b)
        cand_f = jax.lax.bitcast_convert_type(cand, jnp.float32)
        cnt = jnp.sum((h >= cand_f).astype(jnp.int32), axis=0, keepdims=True)
        thr = jnp.where(cnt >= K_WTA, cand, thr)
    thr_f = jax.lax.bitcast_convert_type(thr, jnp.float32)
    s = jnp.where(h >= thr_f, h, 0.0)
    norm = jnp.sqrt(jnp.sum(s * s, axis=0, keepdims=True))
    return s * (1.0 / jnp.maximum(norm, 1e-10))


def _sparsify_body(wd_ref, xT_ref, sT_ref, hprev_ref, hcur_ref):
    sT_ref[...] = (_kwta_normalize(hprev_ref[...]) * 16.0).astype(
        jnp.float8_e4m3fn)
    h = jnp.dot(wd_ref[...], xT_ref[...].astype(jnp.bfloat16),
                preferred_element_type=jnp.float32)
    hcur_ref[...] = jnp.maximum(h, 0.0)
    hprev_ref[...] = hcur_ref[...]


def _retrieve_body(wc_ref, sT_ref, h2T_ref, tot_ref, acc_ref):
    i = pl.program_id(0)
    k = pl.program_id(1)
    nk = pl.num_programs(1)

    part = jnp.dot(wc_ref[...], sT_ref[...],
                   preferred_element_type=jnp.float32)

    @pl.when(k == 0)
    def _init_acc():
        acc_ref[...] = part

    @pl.when(k != 0)
    def _accum():
        acc_ref[...] += part

    @pl.when((i == 0) & (k == 0))
    def _init_tot():
        tot_ref[...] = jnp.zeros((1, 1), jnp.float32)

    @pl.when(k == nk - 1)
    def _finish():
        h2 = jnp.maximum(acc_ref[...] * (1.0 / 1024.0), 0.0)
        tot_ref[...] += jnp.sum(h2).reshape(1, 1)
        h2T_ref[...] = h2.astype(jnp.bfloat16)


def _combine_body(xT_ref, sT_ref, h2T_ref, tot_ref, wu_ref, wg_ref, bg_ref,
                  outT_ref):
    xn = _kwta_normalize(h2T_ref[...].astype(jnp.float32))
    cond = tot_ref[...] < 1e-10
    succ = jnp.where(cond, sT_ref[...].astype(jnp.float32) * (1.0 / 16.0), xn)
    predT = jnp.dot(wu_ref[...], succ.astype(jnp.bfloat16),
                    preferred_element_type=jnp.float32)
    xT = xT_ref[...]
    zT = jnp.dot(wg_ref[...], xT.astype(jnp.bfloat16),
                 preferred_element_type=jnp.float32) + bg_ref[...]
    gT = jax.nn.sigmoid(zT)
    outT_ref[...] = xT + gT * predT


def kernel(x, W_down, W_up, W_gate, b_gate, W_ca3):
    xT = x.reshape(SEQ, D_MODEL).T
    wd = W_down.astype(jnp.bfloat16)
    wc = (W_ca3 * 64.0).astype(jnp.float8_e4m3fn)
    wu = W_up.astype(jnp.bfloat16)
    wg = W_gate.astype(jnp.bfloat16)
    bg = b_gate.reshape(D_MODEL, 1)

    cp = pltpu.CompilerParams(vmem_limit_bytes=63 * 1024 * 1024)

    sT = pl.pallas_call(
        _sparsify_body,
        grid=(GRID + 1,),
        in_specs=[
            pl.BlockSpec((N_CA3, D_MODEL), lambda i: (0, 0)),
            pl.BlockSpec((D_MODEL, TN),
                         lambda i: (0, jnp.minimum(i, GRID - 1))),
        ],
        out_specs=pl.BlockSpec((N_CA3, TN),
                               lambda i: (0, jnp.maximum(i - 1, 0))),
        out_shape=jax.ShapeDtypeStruct((N_CA3, SEQ), jnp.float8_e4m3fn),
        scratch_shapes=[
            pltpu.VMEM((N_CA3, TN), jnp.float32),
            pltpu.VMEM((N_CA3, TN), jnp.float32),
        ],
        compiler_params=cp,
    )(wd, xT)

    KP = 2048
    h2T, tot = pl.pallas_call(
        _retrieve_body,
        grid=(GRID, N_CA3 // KP),
        in_specs=[
            pl.BlockSpec((N_CA3, KP), lambda i, k: (0, k)),
            pl.BlockSpec((KP, TN), lambda i, k: (k, i)),
        ],
        out_specs=[
            pl.BlockSpec((N_CA3, TN), lambda i, k: (0, i)),
            pl.BlockSpec((1, 1), lambda i, k: (0, 0)),
        ],
        out_shape=[
            jax.ShapeDtypeStruct((N_CA3, SEQ), jnp.bfloat16),
            jax.ShapeDtypeStruct((1, 1), jnp.float32),
        ],
        scratch_shapes=[
            pltpu.VMEM((N_CA3, TN), jnp.float32),
        ],
        compiler_params=cp,
    )(wc, sT)

    outT = pl.pallas_call(
        _combine_body,
        grid=(GRID,),
        in_specs=[
            pl.BlockSpec((D_MODEL, TN), lambda i: (0, i)),
            pl.BlockSpec((N_CA3, TN), lambda i: (0, i)),
            pl.BlockSpec((N_CA3, TN), lambda i: (0, i)),
            pl.BlockSpec((1, 1), lambda i: (0, 0)),
            pl.BlockSpec((D_MODEL, N_CA3), lambda i: (0, 0)),
            pl.BlockSpec((D_MODEL, D_MODEL), lambda i: (0, 0)),
            pl.BlockSpec((D_MODEL, 1), lambda i: (0, 0)),
        ],
        out_specs=pl.BlockSpec((D_MODEL, TN), lambda i: (0, i)),
        out_shape=jax.ShapeDtypeStruct((D_MODEL, SEQ), jnp.float32),
        compiler_params=cp,
    )(xT, sT, h2T, tot, wu, wg, bg)

    return outT.T.reshape(1, SEQ, D_MODEL)


# speedup vs baseline: 1.7526x; 1.0113x over previous
import jax
import jax.numpy as jnp
from jax.experimental import pallas as pl
from jax.experimental.pallas import tpu as pltpu

SEQ = 2048
D_MODEL = 768
N_CA3 = 4096
K_WTA = 64
TN = 256
GRID = SEQ // TN


def _kwta_normalize(h):
    ncols = h.shape[1]
    thr = jnp.zeros((1, ncols), dtype=jnp.int32)
    for b in range(30, 19, -1):
        cand = thr | (1 << b)
        cand_f = jax.lax.bitcast_convert_type(cand, jnp.float32)
        cnt = jnp.sum((h >= cand_f).astype(jnp.int32), axis=0, keepdims=True)
        thr = jnp.where(cnt >= K_WTA, cand, thr)
    thr_f = jax.lax.bitcast_convert_type(thr, jnp.float32)
    s = jnp.where(h >= thr_f, h, 0.0)
    norm = jnp.sqrt(jnp.sum(s * s, axis=0, keepdims=True))
    return s * (1.0 / jnp.maximum(norm, 1e-10))


def _sparsify_body(wd_ref, xT_ref, sT_ref, hprev_ref, hcur_ref):
    sT_ref[...] = (_kwta_normalize(hprev_ref[...]) * 16.0).astype(
        jnp.float8_e4m3fn)
    h = jnp.dot(wd_ref[...], xT_ref[...].astype(jnp.bfloat16),
                preferred_element_type=jnp.float32)
    hcur_ref[...] = jnp.maximum(h, 0.0)
    hprev_ref[...] = hcur_ref[...]


def _retrieve_body(wc_ref, sT_ref, h2T_ref, tot_ref, acc_ref):
    i = pl.program_id(0)
    k = pl.program_id(1)
    nk = pl.num_programs(1)

    part = jnp.dot(wc_ref[...], sT_ref[...],
                   preferred_element_type=jnp.float32)

    @pl.when(k == 0)
    def _init_acc():
        acc_ref[...] = part

    @pl.when(k != 0)
    def _accum():
        acc_ref[...] += part

    @pl.when((i == 0) & (k == 0))
    def _init_tot():
        tot_ref[...] = jnp.zeros((1, 1), jnp.float32)

    @pl.when(k == nk - 1)
    def _finish():
        h2 = jnp.maximum(acc_ref[...] * (1.0 / 1024.0), 0.0)
        tot_ref[...] += jnp.sum(h2).reshape(1, 1)
        h2T_ref[...] = h2.astype(jnp.bfloat16)


def _combine_body(xT_ref, sT_ref, h2T_ref, tot_ref, wu_ref, wg_ref, bg_ref,
                  outT_ref, sprev_ref, scur_ref):
    xn = _kwta_normalize(h2T_ref[...].astype(jnp.float32))
    cond = tot_ref[...] < 1e-10
    succ = jnp.where(cond, sT_ref[...].astype(jnp.float32) * (1.0 / 16.0), xn)
    scur_ref[...] = succ.astype(jnp.bfloat16)

    predT = jnp.dot(wu_ref[...], sprev_ref[...],
                    preferred_element_type=jnp.float32)
    xT = xT_ref[...]
    zT = jnp.dot(wg_ref[...], xT.astype(jnp.bfloat16),
                 preferred_element_type=jnp.float32) + bg_ref[...]
    gT = jax.nn.sigmoid(zT)
    outT_ref[...] = xT + gT * predT
    sprev_ref[...] = scur_ref[...]


def kernel(x, W_down, W_up, W_gate, b_gate, W_ca3):
    xT = x.reshape(SEQ, D_MODEL).T
    wd = W_down.astype(jnp.bfloat16)
    wc = (W_ca3 * 64.0).astype(jnp.float8_e4m3fn)
    wu = W_up.astype(jnp.bfloat16)
    wg = W_gate.astype(jnp.bfloat16)
    bg = b_gate.reshape(D_MODEL, 1)

    cp = pltpu.CompilerParams(vmem_limit_bytes=63 * 1024 * 1024)

    sT = pl.pallas_call(
        _sparsify_body,
        grid=(GRID + 1,),
        in_specs=[
            pl.BlockSpec((N_CA3, D_MODEL), lambda i: (0, 0)),
            pl.BlockSpec((D_MODEL, TN),
                         lambda i: (0, jnp.minimum(i, GRID - 1))),
        ],
        out_specs=pl.BlockSpec((N_CA3, TN),
                               lambda i: (0, jnp.maximum(i - 1, 0))),
        out_shape=jax.ShapeDtypeStruct((N_CA3, SEQ), jnp.float8_e4m3fn),
        scratch_shapes=[
            pltpu.VMEM((N_CA3, TN), jnp.float32),
            pltpu.VMEM((N_CA3, TN), jnp.float32),
        ],
        compiler_params=cp,
    )(wd, xT)

    KP = 2048
    h2T, tot = pl.pallas_call(
        _retrieve_body,
        grid=(GRID, N_CA3 // KP),
        in_specs=[
            pl.BlockSpec((N_CA3, KP), lambda i, k: (0, k)),
            pl.BlockSpec((KP, TN), lambda i, k: (k, i)),
        ],
        out_specs=[
            pl.BlockSpec((N_CA3, TN), lambda i, k: (0, i)),
            pl.BlockSpec((1, 1), lambda i, k: (0, 0)),
        ],
        out_shape=[
            jax.ShapeDtypeStruct((N_CA3, SEQ), jnp.bfloat16),
            jax.ShapeDtypeStruct((1, 1), jnp.float32),
        ],
        scratch_shapes=[
            pltpu.VMEM((N_CA3, TN), jnp.float32),
        ],
        compiler_params=cp,
    )(wc, sT)

    outT = pl.pallas_call(
        _combine_body,
        grid=(GRID + 1,),
        in_specs=[
            pl.BlockSpec((D_MODEL, TN),
                         lambda i: (0, jnp.maximum(i - 1, 0))),
            pl.BlockSpec((N_CA3, TN),
                         lambda i: (0, jnp.minimum(i, GRID - 1))),
            pl.BlockSpec((N_CA3, TN),
                         lambda i: (0, jnp.minimum(i, GRID - 1))),
            pl.BlockSpec((1, 1), lambda i: (0, 0)),
            pl.BlockSpec((D_MODEL, N_CA3), lambda i: (0, 0)),
            pl.BlockSpec((D_MODEL, D_MODEL), lambda i: (0, 0)),
            pl.BlockSpec((D_MODEL, 1), lambda i: (0, 0)),
        ],
        out_specs=pl.BlockSpec((D_MODEL, TN),
                               lambda i: (0, jnp.maximum(i - 1, 0))),
        out_shape=jax.ShapeDtypeStruct((D_MODEL, SEQ), jnp.float32),
        scratch_shapes=[
            pltpu.VMEM((N_CA3, TN), jnp.bfloat16),
            pltpu.VMEM((N_CA3, TN), jnp.bfloat16),
        ],
        compiler_params=cp,
    )(xT, sT, h2T, tot, wu, wg, bg)

    return outT.T.reshape(1, SEQ, D_MODEL)


# speedup vs baseline: 1.9072x; 1.0882x over previous
import jax
import jax.numpy as jnp
from jax.experimental import pallas as pl
from jax.experimental.pallas import tpu as pltpu

SEQ = 2048
D_MODEL = 768
N_CA3 = 4096
K_WTA = 64
TN = 256
GRID = SEQ // TN


def _kwta_normalize(h):
    ncols = h.shape[1]
    thr = jnp.zeros((1, ncols), dtype=jnp.int32)
    for b in range(30, 19, -1):
        cand = thr | (1 << b)
        cand_f = jax.lax.bitcast_convert_type(cand, jnp.float32)
        cnt = jnp.sum((h >= cand_f).astype(jnp.int32), axis=0, keepdims=True)
        thr = jnp.where(cnt >= K_WTA, cand, thr)
    thr_f = jax.lax.bitcast_convert_type(thr, jnp.float32)
    s = jnp.where(h >= thr_f, h, 0.0)
    norm = jnp.sqrt(jnp.sum(s * s, axis=0, keepdims=True))
    return s * (1.0 / jnp.maximum(norm, 1e-10))


def _sparsify_body(wd_ref, xT_ref, sT_ref, hprev_ref, hcur_ref):
    sT_ref[...] = (_kwta_normalize(hprev_ref[...]) * 16.0).astype(
        jnp.float8_e4m3fn)
    h = jax.lax.dot_general(wd_ref[...], xT_ref[...].astype(jnp.bfloat16),
                            (((1,), (1,)), ((), ())),
                            preferred_element_type=jnp.float32)
    hcur_ref[...] = jnp.maximum(h, 0.0)
    hprev_ref[...] = hcur_ref[...]


def _retrieve_body(wc_ref, sT_ref, h2T_ref, tot_ref, acc_ref):
    i = pl.program_id(0)
    k = pl.program_id(1)
    nk = pl.num_programs(1)

    part = jnp.dot(wc_ref[...], sT_ref[...],
                   preferred_element_type=jnp.float32)

    @pl.when(k == 0)
    def _init_acc():
        acc_ref[...] = part

    @pl.when(k != 0)
    def _accum():
        acc_ref[...] += part

    @pl.when((i == 0) & (k == 0))
    def _init_tot():
        tot_ref[...] = jnp.zeros((1, 1), jnp.float32)

    @pl.when(k == nk - 1)
    def _finish():
        h2 = jnp.maximum(acc_ref[...] * (1.0 / 1024.0), 0.0)
        tot_ref[...] += jnp.sum(h2).reshape(1, 1)
        h2T_ref[...] = h2.astype(jnp.bfloat16)


def _combine_body(xT_ref, sT_ref, h2T_ref, tot_ref, wu_ref, wg_ref, bg_ref,
                  outT_ref, sprev_ref, scur_ref):
    xn = _kwta_normalize(h2T_ref[...].astype(jnp.float32))
    cond = tot_ref[...] < 1e-10
    succ = jnp.where(cond, sT_ref[...].astype(jnp.float32) * (1.0 / 16.0), xn)
    scur_ref[...] = succ.astype(jnp.bfloat16)

    predT = jnp.dot(wu_ref[...], sprev_ref[...],
                    preferred_element_type=jnp.float32)
    xb = xT_ref[...]
    zT = jax.lax.dot_general(wg_ref[...], xb.astype(jnp.bfloat16),
                             (((1,), (1,)), ((), ())),
                             preferred_element_type=jnp.float32) + bg_ref[...]
    contribT = jax.nn.sigmoid(zT) * predT
    outT_ref[...] = xb + contribT.T
    sprev_ref[...] = scur_ref[...]


def kernel(x, W_down, W_up, W_gate, b_gate, W_ca3):
    xm = x.reshape(SEQ, D_MODEL)
    wd = W_down.astype(jnp.bfloat16)
    wc = (W_ca3 * 64.0).astype(jnp.float8_e4m3fn)
    wu = W_up.astype(jnp.bfloat16)
    wg = W_gate.astype(jnp.bfloat16)
    bg = b_gate.reshape(D_MODEL, 1)

    cp = pltpu.CompilerParams(vmem_limit_bytes=63 * 1024 * 1024)

    sT = pl.pallas_call(
        _sparsify_body,
        grid=(GRID + 1,),
        in_specs=[
            pl.BlockSpec((N_CA3, D_MODEL), lambda i: (0, 0)),
            pl.BlockSpec((TN, D_MODEL),
                         lambda i: (jnp.minimum(i, GRID - 1), 0)),
        ],
        out_specs=pl.BlockSpec((N_CA3, TN),
                               lambda i: (0, jnp.maximum(i - 1, 0))),
        out_shape=jax.ShapeDtypeStruct((N_CA3, SEQ), jnp.float8_e4m3fn),
        scratch_shapes=[
            pltpu.VMEM((N_CA3, TN), jnp.float32),
            pltpu.VMEM((N_CA3, TN), jnp.float32),
        ],
        compiler_params=cp,
    )(wd, xm)

    KP = 2048
    h2T, tot = pl.pallas_call(
        _retrieve_body,
        grid=(GRID, N_CA3 // KP),
        in_specs=[
            pl.BlockSpec((N_CA3, KP), lambda i, k: (0, k)),
            pl.BlockSpec((KP, TN), lambda i, k: (k, i)),
        ],
        out_specs=[
            pl.BlockSpec((N_CA3, TN), lambda i, k: (0, i)),
            pl.BlockSpec((1, 1), lambda i, k: (0, 0)),
        ],
        out_shape=[
            jax.ShapeDtypeStruct((N_CA3, SEQ), jnp.bfloat16),
            jax.ShapeDtypeStruct((1, 1), jnp.float32),
        ],
        scratch_shapes=[
            pltpu.VMEM((N_CA3, TN), jnp.float32),
        ],
        compiler_params=cp,
    )(wc, sT)

    outT = pl.pallas_call(
        _combine_body,
        grid=(GRID + 1,),
        in_specs=[
            pl.BlockSpec((TN, D_MODEL),
                         lambda i: (jnp.maximum(i - 1, 0), 0)),
            pl.BlockSpec((N_CA3, TN),
                         lambda i: (0, jnp.minimum(i, GRID - 1))),
            pl.BlockSpec((N_CA3, TN),
                         lambda i: (0, jnp.minimum(i, GRID - 1))),
            pl.BlockSpec((1, 1), lambda i: (0, 0)),
            pl.BlockSpec((D_MODEL, N_CA3), lambda i: (0, 0)),
            pl.BlockSpec((D_MODEL, D_MODEL), lambda i: (0, 0)),
            pl.BlockSpec((D_MODEL, 1), lambda i: (0, 0)),
        ],
        out_specs=pl.BlockSpec((TN, D_MODEL),
                               lambda i: (jnp.maximum(i - 1, 0), 0)),
        out_shape=jax.ShapeDtypeStruct((SEQ, D_MODEL), jnp.float32),
        scratch_shapes=[
            pltpu.VMEM((N_CA3, TN), jnp.bfloat16),
            pltpu.VMEM((N_CA3, TN), jnp.bfloat16),
        ],
        compiler_params=cp,
    )(xm, sT, h2T, tot, wu, wg, bg)

    return outT.reshape(1, SEQ, D_MODEL)
